# Initial kernel scaffold; baseline (speedup 1.0000x reference)
import jax
import jax.numpy as jnp
from jax.experimental import pallas as pl


def kernel(features, labels, centers):
    raise NotImplementedError("write your pallas kernel here")


# speedup vs baseline: 1.0579x; 1.0579x over previous
import jax
import jax.numpy as jnp
from jax import lax
from jax.experimental import pallas as pl
from jax.experimental.pallas import tpu as pltpu
from jax.experimental.pallas import tpu_sc as plsc
from jax._src.pallas import mpmd as _mpmd

C = 100000
D = 64
B = 16384
CW = 16
NCHUNK = D // CW
ALPHA = 0.5
NTILES = 16
S = B // NTILES
SB = 128
NSB = S // SB


def _body(fch_hbm, labels_hbm, cen4_hbm, loss_hbm, out4_hbm,
          labels_v, cnt_s, idx_v, cen_v, a_v, s_v, z_v, acc_sh, lacc_sh):
    cid = lax.axis_index("c")
    sid = lax.axis_index("s")

    @pl.when(cid == 0)
    def _work():
        base = sid * S
        pltpu.sync_copy(labels_hbm.at[sid], labels_v)

        zeros16 = jnp.zeros((CW,), jnp.float32)
        ones16 = jnp.ones((CW,), jnp.float32)

        def _fill(vec):
            def bd(i, _):
                z_v[i, :] = vec
                return 0
            lax.fori_loop(0, SB, bd, 0)

        _fill(zeros16)
        for k in range(NSB):
            pltpu.sync_copy(z_v, acc_sh.at[labels_v.at[k]])
        plsc.subcore_barrier()
        _fill(ones16)
        for k in range(NSB):
            pltpu.sync_copy(z_v, acc_sh.at[labels_v.at[k]], add=True)
        plsc.subcore_barrier()
        for k in range(NSB):
            pltpu.sync_copy(acc_sh.at[labels_v.at[k]], a_v)

            def inv_bd(i, _, k=k):
                inv = ALPHA / a_v[i, :]
                cnt_s[k * SB + i] = inv[0]
                return 0
            lax.fori_loop(0, SB, inv_bd, 0)
        _fill(zeros16)
        plsc.subcore_barrier()

        loss_acc = zeros16
        for c in range(NCHUNK):
            for k in range(NSB):
                pltpu.sync_copy(z_v, acc_sh.at[labels_v.at[k]])
            plsc.subcore_barrier()
            for k in range(NSB):
                pltpu.sync_copy(fch_hbm.at[c, pl.ds(base + k * SB, SB)], a_v)

                def idx_bd(i, _, c=c, k=k):
                    lbl = labels_v[k, pl.ds(i * CW, CW)]
                    idx_v[pl.ds(i * CW, CW)] = lbl * NCHUNK + c
                    return 0
                lax.fori_loop(0, SB // CW, idx_bd, 0)
                pltpu.sync_copy(cen4_hbm.at[idx_v],
                                cen_v.at[pl.ds(k * SB, SB)])

                def loss_bd(i, acc, k=k):
                    dlt = a_v[i, :] - cen_v[k * SB + i, :]
                    return acc + dlt * dlt
                loss_acc = lax.fori_loop(0, SB, loss_bd, loss_acc)
                pltpu.sync_copy(a_v, acc_sh.at[labels_v.at[k]], add=True)
            plsc.subcore_barrier()
            for k in range(NSB):
                pltpu.sync_copy(acc_sh.at[labels_v.at[k]], s_v)

                def comb_bd(i, _, k=k):
                    s_v[i, :] = ((1.0 - ALPHA) * cen_v[k * SB + i, :]
                                 + cnt_s[k * SB + i] * s_v[i, :])
                    return 0
                lax.fori_loop(0, SB, comb_bd, 0)

                def idx_bd(i, _, c=c, k=k):
                    lbl = labels_v[k, pl.ds(i * CW, CW)]
                    idx_v[pl.ds(i * CW, CW)] = lbl * NCHUNK + c
                    return 0
                lax.fori_loop(0, SB // CW, idx_bd, 0)
                pltpu.sync_copy(s_v, out4_hbm.at[idx_v])
            plsc.subcore_barrier()

        z_v[0, :] = loss_acc
        pltpu.sync_copy(z_v.at[0], lacc_sh.at[sid])
        plsc.subcore_barrier()

        @pl.when(sid == 0)
        def _reduce():
            pltpu.sync_copy(lacc_sh, a_v.at[pl.ds(0, NTILES)])

            def red_bd(i, acc):
                return acc + a_v[i, :]
            tot = lax.fori_loop(0, NTILES, red_bd, zeros16)
            z_v[1, :] = (0.5 / B) * tot
            pltpu.sync_copy(z_v.at[1], loss_hbm)


def kernel(features, labels, centers):
    fch = features.reshape(B, NCHUNK, CW).transpose(1, 0, 2)
    labels3 = labels.reshape(NTILES, NSB, SB)
    cen4 = centers.reshape(C * NCHUNK, CW)
    mesh = plsc.VectorSubcoreMesh(core_axis_name="c", subcore_axis_name="s",
                                  num_cores=2, num_subcores=NTILES)
    call = _mpmd._mpmd_map(
        [(mesh, _body)],
        (jax.ShapeDtypeStruct((CW,), jnp.float32),
         jax.ShapeDtypeStruct((C * NCHUNK, CW), jnp.float32)),
        input_output_aliases={2: 1},
        compiler_params=pltpu.CompilerParams(use_tc_tiling_on_sc=False),
        scratch_types=[
            pltpu.VMEM((NSB, SB), jnp.int32),
            pltpu.SMEM((S,), jnp.float32),
            pltpu.VMEM((SB,), jnp.int32),
            pltpu.VMEM((S, CW), jnp.float32),
            pltpu.VMEM((SB, CW), jnp.float32),
            pltpu.VMEM((SB, CW), jnp.float32),
            pltpu.VMEM((SB, CW), jnp.float32),
            pltpu.VMEM_SHARED((C, CW), jnp.float32),
            pltpu.VMEM_SHARED((NTILES, CW), jnp.float32),
        ],
    )
    loss_vec, out4 = call(fch, labels3, cen4)
    return jnp.sum(loss_vec), out4.reshape(C, D)


# speedup vs baseline: 1.1982x; 1.1326x over previous
import jax
import jax.numpy as jnp
from jax import lax
from jax.experimental import pallas as pl
from jax.experimental.pallas import tpu as pltpu
from jax.experimental.pallas import tpu_sc as plsc
from jax._src.pallas import mpmd as _mpmd

C = 100000
D = 64
B = 16384
CW = 16
NCHUNK = D // CW
ALPHA = 0.5
NTILES = 16
S = B // NTILES
SB = 128
NSB = S // SB


def _body(feat_hbm, labels_hbm, cen4_hbm, loss_hbm, out4_hbm,
          labels_v, cnt_s, idx_v, cen_v, a_v, s_v, z_v, acc_sh, lacc_sh):
    cid = lax.axis_index("c")
    sid = lax.axis_index("s")

    @pl.when(cid == 0)
    def _work():
        base = sid * S
        pltpu.sync_copy(labels_hbm.at[sid], labels_v)

        zeros16 = jnp.zeros((CW,), jnp.float32)
        ones16 = jnp.ones((CW,), jnp.float32)

        def _fill(vec):
            def bd(i, _):
                z_v[i, :] = vec
                return 0
            lax.fori_loop(0, SB, bd, 0, unroll=8)

        _fill(zeros16)
        for k in range(NSB):
            pltpu.sync_copy(z_v, acc_sh.at[labels_v.at[k]])
        plsc.subcore_barrier()
        _fill(ones16)
        for k in range(NSB):
            pltpu.sync_copy(z_v, acc_sh.at[labels_v.at[k]], add=True)
        plsc.subcore_barrier()
        for k in range(NSB):
            pltpu.sync_copy(acc_sh.at[labels_v.at[k]], a_v)

            def inv_bd(i, _, k=k):
                inv = ALPHA / a_v[i, :]
                cnt_s[k * SB + i] = inv[0]
                return 0
            lax.fori_loop(0, SB, inv_bd, 0, unroll=4)
        _fill(zeros16)
        plsc.subcore_barrier()

        loss_acc = zeros16
        for c in range(NCHUNK):
            for k in range(NSB):
                pltpu.sync_copy(z_v, acc_sh.at[labels_v.at[k]])
            plsc.subcore_barrier()
            for k in range(NSB):
                def idx_bd(j, _, c=c, k=k):
                    lbl = labels_v[k, pl.ds(j * CW, CW)]
                    idx_v[k, pl.ds(j * CW, CW)] = lbl * NCHUNK + c
                    return 0
                lax.fori_loop(0, SB // CW, idx_bd, 0, unroll=8)
            for k in range(NSB):
                pltpu.sync_copy(
                    feat_hbm.at[pl.ds(base + k * SB, SB),
                                pl.ds(c * CW, CW)], a_v)
                pltpu.sync_copy(cen4_hbm.at[idx_v.at[k]],
                                cen_v.at[pl.ds(k * SB, SB)])

                def loss_bd(i, acc, k=k):
                    dlt = a_v[i, :] - cen_v[k * SB + i, :]
                    return acc + dlt * dlt
                loss_acc = lax.fori_loop(0, SB, loss_bd, loss_acc, unroll=8)
                pltpu.sync_copy(a_v, acc_sh.at[labels_v.at[k]], add=True)
            plsc.subcore_barrier()
            for k in range(NSB):
                pltpu.sync_copy(acc_sh.at[labels_v.at[k]], s_v)

                def comb_bd(i, _, k=k):
                    s_v[i, :] = ((1.0 - ALPHA) * cen_v[k * SB + i, :]
                                 + cnt_s[k * SB + i] * s_v[i, :])
                    return 0
                lax.fori_loop(0, SB, comb_bd, 0, unroll=8)
                pltpu.sync_copy(s_v, out4_hbm.at[idx_v.at[k]])
            plsc.subcore_barrier()

        z_v[0, :] = loss_acc
        pltpu.sync_copy(z_v.at[0], lacc_sh.at[sid])
        plsc.subcore_barrier()

        @pl.when(sid == 0)
        def _reduce():
            pltpu.sync_copy(lacc_sh, a_v.at[pl.ds(0, NTILES)])

            def red_bd(i, acc):
                return acc + a_v[i, :]
            tot = lax.fori_loop(0, NTILES, red_bd, zeros16)
            z_v[1, :] = (0.5 / B) * tot
            pltpu.sync_copy(z_v.at[1], loss_hbm)


def kernel(features, labels, centers):
    labels3 = labels.reshape(NTILES, NSB, SB)
    cen4 = centers.reshape(C * NCHUNK, CW)
    mesh = plsc.VectorSubcoreMesh(core_axis_name="c", subcore_axis_name="s",
                                  num_cores=2, num_subcores=NTILES)
    call = _mpmd._mpmd_map(
        [(mesh, _body)],
        (jax.ShapeDtypeStruct((CW,), jnp.float32),
         jax.ShapeDtypeStruct((C * NCHUNK, CW), jnp.float32)),
        input_output_aliases={2: 1},
        compiler_params=pltpu.CompilerParams(use_tc_tiling_on_sc=False),
        scratch_types=[
            pltpu.VMEM((NSB, SB), jnp.int32),
            pltpu.SMEM((S,), jnp.float32),
            pltpu.VMEM((NSB, SB), jnp.int32),
            pltpu.VMEM((S, CW), jnp.float32),
            pltpu.VMEM((SB, CW), jnp.float32),
            pltpu.VMEM((SB, CW), jnp.float32),
            pltpu.VMEM((SB, CW), jnp.float32),
            pltpu.VMEM_SHARED((C, CW), jnp.float32),
            pltpu.VMEM_SHARED((NTILES, CW), jnp.float32),
        ],
    )
    loss_vec, out4 = call(features, labels3, cen4)
    return jnp.sum(loss_vec), out4.reshape(C, D)


# speedup vs baseline: 1.4302x; 1.1936x over previous
import jax
import jax.numpy as jnp
from jax import lax
from jax.experimental import pallas as pl
from jax.experimental.pallas import tpu as pltpu
from jax.experimental.pallas import tpu_sc as plsc
from jax._src.pallas import mpmd as _mpmd

C = 100000
D = 64
B = 16384
CW = 16
NCHUNK = D // CW
ALPHA = 0.5
NTILES = 16
S = B // NTILES
SB = 128
NSB = S // SB


def _body(feat_hbm, labels_hbm, cen4_hbm, loss_hbm, out4_hbm,
          labels_v, cnt_s, idx_v, cen_v, a_v, s_v, z_v, acc_sh, lacc_sh,
          sem_fire, sem_a, sem_b):
    cid = lax.axis_index("c")
    sid = lax.axis_index("s")

    @pl.when(cid == 0)
    def _work():
        base = sid * S
        pltpu.sync_copy(labels_hbm.at[sid], labels_v)

        zeros16 = jnp.zeros((CW,), jnp.float32)
        ones16 = jnp.ones((CW,), jnp.float32)

        def _fill(vec):
            def bd(i, _):
                z_v[i, :] = vec
                return 0
            lax.fori_loop(0, SB, bd, 0, unroll=8)

        def _fire_all(mk):
            ds = [mk(k) for k in range(NSB)]
            for d in ds:
                d.wait()

        _fill(zeros16)
        _fire_all(lambda k: pltpu.async_copy(
            z_v, acc_sh.at[labels_v.at[k]], sem_fire))
        plsc.subcore_barrier()
        _fill(ones16)
        _fire_all(lambda k: pltpu.async_copy(
            z_v, acc_sh.at[labels_v.at[k]], sem_fire, add=True))
        plsc.subcore_barrier()
        gd = [None] * NSB
        gd[0] = pltpu.async_copy(acc_sh.at[labels_v.at[0]],
                                 a_v.at[0], sem_a.at[0])
        for k in range(NSB):
            if k + 1 < NSB:
                gd[k + 1] = pltpu.async_copy(
                    acc_sh.at[labels_v.at[k + 1]],
                    a_v.at[(k + 1) % 2], sem_a.at[(k + 1) % 2])
            gd[k].wait()

            def inv_bd(i, _, k=k):
                inv = ALPHA / a_v[k % 2, i, :]
                cnt_s[k * SB + i] = inv[0]
                return 0
            lax.fori_loop(0, SB, inv_bd, 0, unroll=4)
        _fill(zeros16)
        plsc.subcore_barrier()

        loss_acc = zeros16
        for c in range(NCHUNK):
            for k in range(NSB):
                def idx_bd(j, _, c=c, k=k):
                    lbl = labels_v[k, pl.ds(j * CW, CW)]
                    idx_v[k, pl.ds(j * CW, CW)] = lbl * NCHUNK + c
                    return 0
                lax.fori_loop(0, SB // CW, idx_bd, 0, unroll=8)
            _fire_all(lambda k: pltpu.async_copy(
                z_v, acc_sh.at[labels_v.at[k]], sem_fire))
            cd = [pltpu.async_copy(cen4_hbm.at[idx_v.at[k]],
                                   cen_v.at[pl.ds(k * SB, SB)], sem_fire)
                  for k in range(NSB)]
            plsc.subcore_barrier()
            ld = [None] * NSB
            ad = [None] * NSB
            ld[0] = pltpu.async_copy(
                feat_hbm.at[pl.ds(base, SB), pl.ds(c * CW, CW)],
                a_v.at[0], sem_a.at[0])
            for k in range(NSB):
                if k >= 1:
                    ad[k - 1].wait()
                if k + 1 < NSB:
                    ld[k + 1] = pltpu.async_copy(
                        feat_hbm.at[pl.ds(base + (k + 1) * SB, SB),
                                    pl.ds(c * CW, CW)],
                        a_v.at[(k + 1) % 2], sem_a.at[(k + 1) % 2])
                ld[k].wait()
                if k == 0:
                    for d in cd:
                        d.wait()

                def loss_bd(i, acc, k=k):
                    dlt = a_v[k % 2, i, :] - cen_v[k * SB + i, :]
                    return acc + dlt * dlt
                loss_acc = lax.fori_loop(0, SB, loss_bd, loss_acc, unroll=8)
                ad[k] = pltpu.async_copy(
                    a_v.at[k % 2], acc_sh.at[labels_v.at[k]],
                    sem_b.at[k % 2], add=True)
            ad[NSB - 1].wait()
            plsc.subcore_barrier()
            sd = [None] * NSB
            od = [None] * NSB
            sd[0] = pltpu.async_copy(acc_sh.at[labels_v.at[0]],
                                     s_v.at[0], sem_a.at[0])
            for k in range(NSB):
                if k >= 1:
                    od[k - 1].wait()
                if k + 1 < NSB:
                    sd[k + 1] = pltpu.async_copy(
                        acc_sh.at[labels_v.at[k + 1]],
                        s_v.at[(k + 1) % 2], sem_a.at[(k + 1) % 2])
                sd[k].wait()

                def comb_bd(i, _, k=k):
                    s_v[k % 2, i, :] = ((1.0 - ALPHA) * cen_v[k * SB + i, :]
                                        + cnt_s[k * SB + i] * s_v[k % 2, i, :])
                    return 0
                lax.fori_loop(0, SB, comb_bd, 0, unroll=8)
                od[k] = pltpu.async_copy(
                    s_v.at[k % 2], out4_hbm.at[idx_v.at[k]], sem_b.at[k % 2])
            od[NSB - 1].wait()
            plsc.subcore_barrier()

        z_v[0, :] = loss_acc
        pltpu.sync_copy(z_v.at[0], lacc_sh.at[sid])
        plsc.subcore_barrier()

        @pl.when(sid == 0)
        def _reduce():
            pltpu.sync_copy(lacc_sh, a_v.at[0].at[pl.ds(0, NTILES)])

            def red_bd(i, acc):
                return acc + a_v[0, i, :]
            tot = lax.fori_loop(0, NTILES, red_bd, zeros16)
            z_v[1, :] = (0.5 / B) * tot
            pltpu.sync_copy(z_v.at[1], loss_hbm)


def kernel(features, labels, centers):
    labels3 = labels.reshape(NTILES, NSB, SB)
    mesh = plsc.VectorSubcoreMesh(core_axis_name="c", subcore_axis_name="s",
                                  num_cores=2, num_subcores=NTILES)
    call = _mpmd._mpmd_map(
        [(mesh, _body)],
        (jax.ShapeDtypeStruct((CW,), jnp.float32),
         jax.ShapeDtypeStruct((C * NCHUNK, CW), jnp.float32)),
        input_output_aliases={2: 1},
        compiler_params=pltpu.CompilerParams(use_tc_tiling_on_sc=False),
        scratch_types=[
            pltpu.VMEM((NSB, SB), jnp.int32),
            pltpu.SMEM((S,), jnp.float32),
            pltpu.VMEM((NSB, SB), jnp.int32),
            pltpu.VMEM((S, CW), jnp.float32),
            pltpu.VMEM((2, SB, CW), jnp.float32),
            pltpu.VMEM((2, SB, CW), jnp.float32),
            pltpu.VMEM((SB, CW), jnp.float32),
            pltpu.VMEM_SHARED((C, CW), jnp.float32),
            pltpu.VMEM_SHARED((NTILES, CW), jnp.float32),
            pltpu.SemaphoreType.DMA,
            pltpu.SemaphoreType.DMA((2,)),
            pltpu.SemaphoreType.DMA((2,)),
        ],
    )
    loss_vec, out4 = call(features, labels3, centers.reshape(C * NCHUNK, CW))
    return jnp.sum(loss_vec), out4.reshape(C, D)


# speedup vs baseline: 1.6687x; 1.1668x over previous
import jax
import jax.numpy as jnp
from jax import lax
from jax.experimental import pallas as pl
from jax.experimental.pallas import tpu as pltpu
from jax.experimental.pallas import tpu_sc as plsc
from jax._src.pallas import mpmd as _mpmd

C = 100000
D = 64
B = 16384
CW = 16
NCHUNK = D // CW
ALPHA = 0.5
NTILES = 16
S = B // NTILES
SB = 128
NSB = S // SB


def _body(feat_hbm, labels_hbm, cen4_hbm, loss_hbm, out4_hbm,
          labels_v, cnt_s, idx_v, cen_v, a_v, s_v, z_v, acc_sh, lacc_sh,
          sem_fire, sem_a, sem_b):
    cid = lax.axis_index("c")
    sid = lax.axis_index("s")

    if True:
        base = sid * S
        pltpu.sync_copy(labels_hbm.at[sid], labels_v)

        zeros16 = jnp.zeros((CW,), jnp.float32)
        ones16 = jnp.ones((CW,), jnp.float32)

        def _fill(vec):
            def bd(i, _):
                z_v[i, :] = vec
                return 0
            lax.fori_loop(0, SB, bd, 0, unroll=8)

        def _fire_all(mk):
            ds = [mk(k) for k in range(NSB)]
            for d in ds:
                d.wait()

        _fill(zeros16)
        _fire_all(lambda k: pltpu.async_copy(
            z_v, acc_sh.at[labels_v.at[k]], sem_fire))
        plsc.subcore_barrier()
        _fill(ones16)
        _fire_all(lambda k: pltpu.async_copy(
            z_v, acc_sh.at[labels_v.at[k]], sem_fire, add=True))
        plsc.subcore_barrier()
        gd = [None] * NSB
        gd[0] = pltpu.async_copy(acc_sh.at[labels_v.at[0]],
                                 a_v.at[0], sem_a.at[0])
        for k in range(NSB):
            if k + 1 < NSB:
                gd[k + 1] = pltpu.async_copy(
                    acc_sh.at[labels_v.at[k + 1]],
                    a_v.at[(k + 1) % 2], sem_a.at[(k + 1) % 2])
            gd[k].wait()

            def inv_bd(i, _, k=k):
                inv = ALPHA / a_v[k % 2, i, :]
                cnt_s[k * SB + i] = inv[0]
                return 0
            lax.fori_loop(0, SB, inv_bd, 0, unroll=4)
        _fill(zeros16)
        plsc.subcore_barrier()

        loss_acc = zeros16
        for cc in range(NCHUNK // 2):
            c = cid * (NCHUNK // 2) + cc
            for k in range(NSB):
                def idx_bd(j, _, c=c, k=k):
                    lbl = labels_v[k, pl.ds(j * CW, CW)]
                    idx_v[k, pl.ds(j * CW, CW)] = lbl * NCHUNK + c
                    return 0
                lax.fori_loop(0, SB // CW, idx_bd, 0, unroll=8)
            _fire_all(lambda k: pltpu.async_copy(
                z_v, acc_sh.at[labels_v.at[k]], sem_fire))
            cd = [pltpu.async_copy(cen4_hbm.at[idx_v.at[k]],
                                   cen_v.at[pl.ds(k * SB, SB)], sem_fire)
                  for k in range(NSB)]
            plsc.subcore_barrier()
            ld = [None] * NSB
            ad = [None] * NSB
            ld[0] = pltpu.async_copy(
                feat_hbm.at[pl.ds(base, SB), pl.ds(c * CW, CW)],
                a_v.at[0], sem_a.at[0])
            for k in range(NSB):
                if k >= 1:
                    ad[k - 1].wait()
                if k + 1 < NSB:
                    ld[k + 1] = pltpu.async_copy(
                        feat_hbm.at[pl.ds(base + (k + 1) * SB, SB),
                                    pl.ds(c * CW, CW)],
                        a_v.at[(k + 1) % 2], sem_a.at[(k + 1) % 2])
                ld[k].wait()
                if k == 0:
                    for d in cd:
                        d.wait()

                def loss_bd(i, acc, k=k):
                    dlt = a_v[k % 2, i, :] - cen_v[k * SB + i, :]
                    return acc + dlt * dlt
                loss_acc = lax.fori_loop(0, SB, loss_bd, loss_acc, unroll=8)
                ad[k] = pltpu.async_copy(
                    a_v.at[k % 2], acc_sh.at[labels_v.at[k]],
                    sem_b.at[k % 2], add=True)
            ad[NSB - 1].wait()
            plsc.subcore_barrier()
            sd = [None] * NSB
            od = [None] * NSB
            sd[0] = pltpu.async_copy(acc_sh.at[labels_v.at[0]],
                                     s_v.at[0], sem_a.at[0])
            for k in range(NSB):
                if k >= 1:
                    od[k - 1].wait()
                if k + 1 < NSB:
                    sd[k + 1] = pltpu.async_copy(
                        acc_sh.at[labels_v.at[k + 1]],
                        s_v.at[(k + 1) % 2], sem_a.at[(k + 1) % 2])
                sd[k].wait()

                def comb_bd(i, _, k=k):
                    s_v[k % 2, i, :] = ((1.0 - ALPHA) * cen_v[k * SB + i, :]
                                        + cnt_s[k * SB + i] * s_v[k % 2, i, :])
                    return 0
                lax.fori_loop(0, SB, comb_bd, 0, unroll=8)
                od[k] = pltpu.async_copy(
                    s_v.at[k % 2], out4_hbm.at[idx_v.at[k]], sem_b.at[k % 2])
            od[NSB - 1].wait()
            plsc.subcore_barrier()

        z_v[0, :] = loss_acc
        pltpu.sync_copy(z_v.at[0], lacc_sh.at[sid])
        plsc.subcore_barrier()

        @pl.when(sid == 0)
        def _reduce():
            pltpu.sync_copy(lacc_sh, a_v.at[0].at[pl.ds(0, NTILES)])

            def red_bd(i, acc):
                return acc + a_v[0, i, :]
            tot = lax.fori_loop(0, NTILES, red_bd, zeros16)
            z_v[1, :] = (0.5 / B) * tot
            pltpu.sync_copy(z_v.at[1], loss_hbm.at[cid])


def kernel(features, labels, centers):
    labels3 = labels.reshape(NTILES, NSB, SB)
    mesh = plsc.VectorSubcoreMesh(core_axis_name="c", subcore_axis_name="s",
                                  num_cores=2, num_subcores=NTILES)
    call = _mpmd._mpmd_map(
        [(mesh, _body)],
        (jax.ShapeDtypeStruct((2, CW), jnp.float32),
         jax.ShapeDtypeStruct((C * NCHUNK, CW), jnp.float32)),
        input_output_aliases={2: 1},
        compiler_params=pltpu.CompilerParams(use_tc_tiling_on_sc=False),
        scratch_types=[
            pltpu.VMEM((NSB, SB), jnp.int32),
            pltpu.SMEM((S,), jnp.float32),
            pltpu.VMEM((NSB, SB), jnp.int32),
            pltpu.VMEM((S, CW), jnp.float32),
            pltpu.VMEM((2, SB, CW), jnp.float32),
            pltpu.VMEM((2, SB, CW), jnp.float32),
            pltpu.VMEM((SB, CW), jnp.float32),
            pltpu.VMEM_SHARED((C, CW), jnp.float32),
            pltpu.VMEM_SHARED((NTILES, CW), jnp.float32),
            pltpu.SemaphoreType.DMA,
            pltpu.SemaphoreType.DMA((2,)),
            pltpu.SemaphoreType.DMA((2,)),
        ],
    )
    loss_vec, out4 = call(features, labels3, centers.reshape(C * NCHUNK, CW))
    return jnp.sum(loss_vec), out4.reshape(C, D)


# speedup vs baseline: 1.6857x; 1.0102x over previous
import jax
import jax.numpy as jnp
from jax import lax
from jax.experimental import pallas as pl
from jax.experimental.pallas import tpu as pltpu
from jax.experimental.pallas import tpu_sc as plsc
from jax._src.pallas import mpmd as _mpmd

C = 100000
D = 64
B = 16384
CW = 16
NCHUNK = D // CW
ALPHA = 0.5
NTILES = 16
S = B // NTILES
SB = 128
NSB = S // SB


def _body(feat_hbm, labels_hbm, cen4_hbm, loss_hbm, out4_hbm,
          labels_v, cnt_s, idx_v, cen_v, a_v, s_v, z_v, acc_sh, lacc_sh,
          sem_fire, sem_a, sem_b, sem_c):
    cid = lax.axis_index("c")
    sid = lax.axis_index("s")

    if True:
        base = sid * S
        pltpu.sync_copy(labels_hbm.at[sid], labels_v)

        zeros16 = jnp.zeros((CW,), jnp.float32)
        ones16 = jnp.ones((CW,), jnp.float32)

        def _compute_idx(c):
            for k in range(NSB):
                def idx_bd(j, _, c=c, k=k):
                    lbl = labels_v[k, pl.ds(j * CW, CW)]
                    idx_v[k, pl.ds(j * CW, CW)] = lbl * NCHUNK + c
                    return 0
                lax.fori_loop(0, SB // CW, idx_bd, 0, unroll=8)

        c0 = cid * (NCHUNK // 2)
        _compute_idx(c0)
        cd = [pltpu.async_copy(cen4_hbm.at[idx_v.at[k]],
                               cen_v.at[pl.ds(k * SB, SB)], sem_c)
              for k in range(NSB)]

        def _fill(vec):
            def bd(i, _):
                z_v[i, :] = vec
                return 0
            lax.fori_loop(0, SB, bd, 0, unroll=8)

        def _fire_all(mk):
            ds = [mk(k) for k in range(NSB)]
            for d in ds:
                d.wait()

        _fill(zeros16)
        _fire_all(lambda k: pltpu.async_copy(
            z_v, acc_sh.at[labels_v.at[k]], sem_fire))
        plsc.subcore_barrier()
        _fill(ones16)
        _fire_all(lambda k: pltpu.async_copy(
            z_v, acc_sh.at[labels_v.at[k]], sem_fire, add=True))
        plsc.subcore_barrier()
        gd = [None] * NSB
        gd[0] = pltpu.async_copy(acc_sh.at[labels_v.at[0]],
                                 a_v.at[0], sem_a.at[0])
        for k in range(NSB):
            if k + 1 < NSB:
                gd[k + 1] = pltpu.async_copy(
                    acc_sh.at[labels_v.at[k + 1]],
                    a_v.at[(k + 1) % 2], sem_a.at[(k + 1) % 2])
            gd[k].wait()

            def inv_bd(i, _, k=k):
                inv = ALPHA / a_v[k % 2, i, :]
                cnt_s[k * SB + i] = inv[0]
                return 0
            lax.fori_loop(0, SB, inv_bd, 0, unroll=4)
        _fill(zeros16)
        plsc.subcore_barrier()

        loss_acc = zeros16
        for cc in range(NCHUNK // 2):
            c = cid * (NCHUNK // 2) + cc
            if cc > 0:
                _compute_idx(c)
                cd = [pltpu.async_copy(cen4_hbm.at[idx_v.at[k]],
                                       cen_v.at[pl.ds(k * SB, SB)], sem_c)
                      for k in range(NSB)]
            _fire_all(lambda k: pltpu.async_copy(
                z_v, acc_sh.at[labels_v.at[k]], sem_fire))
            plsc.subcore_barrier()
            ld = [None] * NSB
            ad = [None] * NSB
            ld[0] = pltpu.async_copy(
                feat_hbm.at[pl.ds(base, SB), pl.ds(c * CW, CW)],
                a_v.at[0], sem_a.at[0])
            for k in range(NSB):
                if k >= 1:
                    ad[k - 1].wait()
                if k + 1 < NSB:
                    ld[k + 1] = pltpu.async_copy(
                        feat_hbm.at[pl.ds(base + (k + 1) * SB, SB),
                                    pl.ds(c * CW, CW)],
                        a_v.at[(k + 1) % 2], sem_a.at[(k + 1) % 2])
                ld[k].wait()
                if k == 0:
                    for d in cd:
                        d.wait()

                def loss_bd(i, acc, k=k):
                    dlt = a_v[k % 2, i, :] - cen_v[k * SB + i, :]
                    return acc + dlt * dlt
                loss_acc = lax.fori_loop(0, SB, loss_bd, loss_acc, unroll=8)
                ad[k] = pltpu.async_copy(
                    a_v.at[k % 2], acc_sh.at[labels_v.at[k]],
                    sem_b.at[k % 2], add=True)
            ad[NSB - 1].wait()
            plsc.subcore_barrier()
            sd = [None] * NSB
            od = [None] * NSB
            sd[0] = pltpu.async_copy(acc_sh.at[labels_v.at[0]],
                                     s_v.at[0], sem_a.at[0])
            for k in range(NSB):
                if k >= 1:
                    od[k - 1].wait()
                if k + 1 < NSB:
                    sd[k + 1] = pltpu.async_copy(
                        acc_sh.at[labels_v.at[k + 1]],
                        s_v.at[(k + 1) % 2], sem_a.at[(k + 1) % 2])
                sd[k].wait()

                def comb_bd(i, _, k=k):
                    s_v[k % 2, i, :] = ((1.0 - ALPHA) * cen_v[k * SB + i, :]
                                        + cnt_s[k * SB + i] * s_v[k % 2, i, :])
                    return 0
                lax.fori_loop(0, SB, comb_bd, 0, unroll=8)
                od[k] = pltpu.async_copy(
                    s_v.at[k % 2], out4_hbm.at[idx_v.at[k]], sem_b.at[k % 2])
            od[NSB - 1].wait()
            plsc.subcore_barrier()

        z_v[0, :] = loss_acc
        pltpu.sync_copy(z_v.at[0], lacc_sh.at[sid])
        plsc.subcore_barrier()

        @pl.when(sid == 0)
        def _reduce():
            pltpu.sync_copy(lacc_sh, a_v.at[0].at[pl.ds(0, NTILES)])

            def red_bd(i, acc):
                return acc + a_v[0, i, :]
            tot = lax.fori_loop(0, NTILES, red_bd, zeros16)
            z_v[1, :] = (0.5 / B) * tot
            pltpu.sync_copy(z_v.at[1], loss_hbm.at[cid])


def kernel(features, labels, centers):
    labels3 = labels.reshape(NTILES, NSB, SB)
    mesh = plsc.VectorSubcoreMesh(core_axis_name="c", subcore_axis_name="s",
                                  num_cores=2, num_subcores=NTILES)
    call = _mpmd._mpmd_map(
        [(mesh, _body)],
        (jax.ShapeDtypeStruct((2, CW), jnp.float32),
         jax.ShapeDtypeStruct((C * NCHUNK, CW), jnp.float32)),
        input_output_aliases={2: 1},
        compiler_params=pltpu.CompilerParams(use_tc_tiling_on_sc=False),
        scratch_types=[
            pltpu.VMEM((NSB, SB), jnp.int32),
            pltpu.SMEM((S,), jnp.float32),
            pltpu.VMEM((NSB, SB), jnp.int32),
            pltpu.VMEM((S, CW), jnp.float32),
            pltpu.VMEM((2, SB, CW), jnp.float32),
            pltpu.VMEM((2, SB, CW), jnp.float32),
            pltpu.VMEM((SB, CW), jnp.float32),
            pltpu.VMEM_SHARED((C, CW), jnp.float32),
            pltpu.VMEM_SHARED((NTILES, CW), jnp.float32),
            pltpu.SemaphoreType.DMA,
            pltpu.SemaphoreType.DMA((2,)),
            pltpu.SemaphoreType.DMA((2,)),
            pltpu.SemaphoreType.DMA,
        ],
    )
    loss_vec, out4 = call(features, labels3, centers.reshape(C * NCHUNK, CW))
    return jnp.sum(loss_vec), out4.reshape(C, D)
